# baseline (speedup 1.0000x reference)
import dataclasses

import numpy as np

import jax
import jax.numpy as jnp
from jax import lax
from jax.experimental import pallas as pl
from jax.experimental.pallas import tpu as pltpu
from jax.experimental.pallas import tpu_sc as plsc

HOP = 128
FFT = 4096
NUM_LABELS = 128
SEQ = 32768
PAD = FFT // 2
OUT_T = 257
BATCH = 8
WORKERS_PER_BATCH = 4
NUM_WORKERS = BATCH * WORKERS_PER_BATCH
SPAN = 12288
RES_PAD = 80
LANES = 16
GROUPS = HOP // LANES

_g = np.arange(OUT_T)
_q = np.minimum(_g // 64, 3)
_slot = _g - 64 * _q
_GATHER = (np.arange(BATCH)[:, None] * WORKERS_PER_BATCH + _q[None, :]) \
    * RES_PAD + _slot[None, :]


def _sc_compiler_params():
    cp = pltpu.CompilerParams()
    if "needs_layout_passes" in pltpu.CompilerParams.__dataclass_fields__:
        cp = dataclasses.replace(cp, needs_layout_passes=False)
    return cp


def _label_argmax_sc(lbl_flat):
    mesh = plsc.VectorSubcoreMesh(core_axis_name="c", subcore_axis_name="s")

    @pl.kernel(
        out_type=jax.ShapeDtypeStruct((NUM_WORKERS, RES_PAD), jnp.int32),
        mesh=mesh,
        scratch_types=[
            pltpu.VMEM((SPAN,), jnp.int32),
            pltpu.VMEM((NUM_LABELS,), jnp.int32),
            pltpu.VMEM((RES_PAD,), jnp.int32),
        ],
        compiler_params=_sc_compiler_params(),
    )
    def k(lbl_hbm, out_hbm, lbl_v, hist_v, res_v):
        wid = lax.axis_index("c") * 16 + lax.axis_index("s")
        b = wid // WORKERS_PER_BATCH
        q = wid % WORKERS_PER_BATCH

        s_off = jnp.clip(q * 8192 - PAD, 0, SEQ - SPAN)
        dma_off = pl.multiple_of(b * SEQ + s_off, PAD)
        pltpu.sync_copy(lbl_hbm.at[pl.ds(dma_off, SPAN)], lbl_v)

        lane_iota = lax.iota(jnp.int32, LANES)
        top_lane = lane_iota == LANES - 1
        lane0 = lane_iota == 0
        ones = jnp.full((LANES,), 1, jnp.int32)
        neg_ones = jnp.full((LANES,), -1, jnp.int32)
        is_q0 = q == 0
        key_offs = [
            jnp.full((LANES,), NUM_LABELS - 1 - j * LANES, jnp.int32) - lane_iota
            for j in range(GROUPS)
        ]

        zeros16 = jnp.zeros((LANES,), jnp.int32)
        for j in range(NUM_LABELS // LANES):
            hist_v[pl.ds(j * LANES, LANES)] = zeros16

        def scatter_batch(offs, sign):
            vs = [lbl_v[pl.ds(off, LANES)] for off in offs]
            scans = [plsc.scan_count(v) for v in vs]
            for v, (cnt, last) in zip(vs, scans):
                plsc.addupdate_scatter(
                    hist_v, [v], cnt if sign > 0 else -cnt, mask=last)
            return vs

        def argmax_store(slot):
            m = hist_v[pl.ds(0, LANES)] * NUM_LABELS + key_offs[0]
            for j in range(1, GROUPS):
                key = hist_v[pl.ds(j * LANES, LANES)] * NUM_LABELS + key_offs[j]
                m = jnp.maximum(m, key)
            s = lax.sort(m)
            best = (NUM_LABELS - 1) - (s & (NUM_LABELS - 1))
            idx = jnp.full((LANES,), slot, jnp.int32)
            plsc.store_scatter(res_v, [idx], best, mask=top_lane)

        b_init = q * 8192 - PAD - s_off

        @pl.loop(0, FFT // HOP)
        def _(c):
            offs = []
            for j in range(GROUPS):
                g = c * GROUPS + j
                offs.append(jnp.where(is_q0, (g % (PAD // LANES)) * LANES,
                                      b_init + g * LANES))
            scatter_batch(offs, 1)

        mask_q0 = lane0 & is_q0
        v_lo = lbl_v[pl.ds(0, LANES)]
        v_hi = lbl_v[pl.ds(PAD, LANES)]
        plsc.addupdate_scatter(hist_v, [v_lo], neg_ones, mask=mask_q0)
        plsc.addupdate_scatter(hist_v, [v_hi], ones, mask=mask_q0)

        argmax_store(0)

        @pl.loop(0, WIN_SLIDES := 64)
        def _(i):
            c_sub = q * 64 + i
            c_add = c_sub + 32
            pad_l = c_sub < 16
            pad_r = c_add >= 272
            interior_sub = c_sub * HOP - PAD - s_off
            interior_add = c_add * HOP - PAD - s_off
            ab_sub = jnp.where(pad_l, 1920 - HOP * c_sub, interior_sub)
            ab_add = jnp.where(pad_r, 46976 - HOP * c_add, interior_add)

            subs = [lbl_v[pl.ds(ab_sub + j * LANES, LANES)]
                    for j in range(GROUPS)]
            adds = [lbl_v[pl.ds(ab_add + j * LANES, LANES)]
                    for j in range(GROUPS)]
            x_sub = lbl_v[pl.ds(ab_sub + HOP, LANES)]
            x_add = lbl_v[pl.ds(ab_add - LANES, LANES)]
            scans_s = [plsc.scan_count(v) for v in subs]
            scans_a = [plsc.scan_count(v) for v in adds]
            for v, (cnt, last) in zip(subs, scans_s):
                plsc.addupdate_scatter(hist_v, [v], -cnt, mask=last)
            for v, (cnt, last) in zip(adds, scans_a):
                plsc.addupdate_scatter(hist_v, [v], cnt, mask=last)

            m_l = lane0 & pad_l
            m_r = top_lane & pad_r
            plsc.addupdate_scatter(hist_v, [subs[0]], ones, mask=m_l)
            plsc.addupdate_scatter(hist_v, [x_sub], neg_ones, mask=m_l)
            plsc.addupdate_scatter(hist_v, [x_add], ones, mask=m_r)
            plsc.addupdate_scatter(hist_v, [adds[7]], neg_ones, mask=m_r)

            argmax_store(i + 1)

        pltpu.sync_copy(res_v, out_hbm.at[wid])

    return k(lbl_flat)


def kernel(lbl, W):
    del W
    res = _label_argmax_sc(lbl.reshape(-1))
    return jnp.take(res.reshape(-1), jnp.asarray(_GATHER, dtype=jnp.int32))


# speedup vs baseline: 1.2901x; 1.2901x over previous
import dataclasses

import numpy as np

import jax
import jax.numpy as jnp
from jax import lax
from jax.experimental import pallas as pl
from jax.experimental.pallas import tpu as pltpu
from jax.experimental.pallas import tpu_sc as plsc

HOP = 128
FFT = 4096
NUM_LABELS = 128
SEQ = 32768
PAD = FFT // 2
OUT_T = 257
BATCH = 8
WORKERS_PER_BATCH = 4
NUM_WORKERS = BATCH * WORKERS_PER_BATCH
SPAN = 12288
RES_PAD = 80
LANES = 16
GROUPS = HOP // LANES

def _sc_compiler_params():
    cp = pltpu.CompilerParams()
    if "needs_layout_passes" in pltpu.CompilerParams.__dataclass_fields__:
        cp = dataclasses.replace(cp, needs_layout_passes=False)
    return cp


def _label_argmax_sc(lbl_flat):
    mesh = plsc.VectorSubcoreMesh(core_axis_name="c", subcore_axis_name="s")

    @pl.kernel(
        out_type=jax.ShapeDtypeStruct((NUM_WORKERS, RES_PAD), jnp.int32),
        mesh=mesh,
        scratch_types=[
            pltpu.VMEM((SPAN,), jnp.int32),
            pltpu.VMEM((NUM_LABELS,), jnp.int32),
            pltpu.VMEM((RES_PAD,), jnp.int32),
        ],
        compiler_params=_sc_compiler_params(),
    )
    def k(lbl_hbm, out_hbm, lbl_v, hist_v, res_v):
        wid = lax.axis_index("c") * 16 + lax.axis_index("s")
        b = wid // WORKERS_PER_BATCH
        q = wid % WORKERS_PER_BATCH

        s_off = jnp.clip(q * 8192 - PAD, 0, SEQ - SPAN)
        dma_off = pl.multiple_of(b * SEQ + s_off, PAD)
        pltpu.sync_copy(lbl_hbm.at[pl.ds(dma_off, SPAN)], lbl_v)

        lane_iota = lax.iota(jnp.int32, LANES)
        top_lane = lane_iota == LANES - 1
        lane0 = lane_iota == 0
        ones = jnp.full((LANES,), 1, jnp.int32)
        neg_ones = jnp.full((LANES,), -1, jnp.int32)
        is_q0 = q == 0
        key_offs = [
            jnp.full((LANES,), NUM_LABELS - 1 - j * LANES, jnp.int32) - lane_iota
            for j in range(GROUPS)
        ]

        zeros16 = jnp.zeros((LANES,), jnp.int32)
        for j in range(NUM_LABELS // LANES):
            hist_v[pl.ds(j * LANES, LANES)] = zeros16

        def scatter_batch(offs, sign):
            vs = [lbl_v[pl.ds(off, LANES)] for off in offs]
            scans = [plsc.scan_count(v) for v in vs]
            for v, (cnt, last) in zip(vs, scans):
                plsc.addupdate_scatter(
                    hist_v, [v], cnt if sign > 0 else -cnt, mask=last)
            return vs

        def argmax_store(slot):
            m = hist_v[pl.ds(0, LANES)] * NUM_LABELS + key_offs[0]
            for j in range(1, GROUPS):
                key = hist_v[pl.ds(j * LANES, LANES)] * NUM_LABELS + key_offs[j]
                m = jnp.maximum(m, key)
            s = lax.sort(m)
            best = (NUM_LABELS - 1) - (s & (NUM_LABELS - 1))
            idx = jnp.full((LANES,), slot, jnp.int32)
            plsc.store_scatter(res_v, [idx], best, mask=top_lane)

        b_init = q * 8192 - PAD - s_off

        @pl.loop(0, FFT // HOP)
        def _(c):
            offs = []
            for j in range(GROUPS):
                g = c * GROUPS + j
                offs.append(jnp.where(is_q0, (g % (PAD // LANES)) * LANES,
                                      b_init + g * LANES))
            scatter_batch(offs, 1)

        mask_q0 = lane0 & is_q0
        v_lo = lbl_v[pl.ds(0, LANES)]
        v_hi = lbl_v[pl.ds(PAD, LANES)]
        plsc.addupdate_scatter(hist_v, [v_lo], neg_ones, mask=mask_q0)
        plsc.addupdate_scatter(hist_v, [v_hi], ones, mask=mask_q0)

        argmax_store(0)

        @pl.loop(0, WIN_SLIDES := 64)
        def _(i):
            c_sub = q * 64 + i
            c_add = c_sub + 32
            pad_l = c_sub < 16
            pad_r = c_add >= 272
            interior_sub = c_sub * HOP - PAD - s_off
            interior_add = c_add * HOP - PAD - s_off
            ab_sub = jnp.where(pad_l, 1920 - HOP * c_sub, interior_sub)
            ab_add = jnp.where(pad_r, 46976 - HOP * c_add, interior_add)

            subs = [lbl_v[pl.ds(ab_sub + j * LANES, LANES)]
                    for j in range(GROUPS)]
            adds = [lbl_v[pl.ds(ab_add + j * LANES, LANES)]
                    for j in range(GROUPS)]
            x_sub = lbl_v[pl.ds(ab_sub + HOP, LANES)]
            x_add = lbl_v[pl.ds(ab_add - LANES, LANES)]
            scans_s = [plsc.scan_count(v) for v in subs]
            scans_a = [plsc.scan_count(v) for v in adds]
            for v, (cnt, last) in zip(subs, scans_s):
                plsc.addupdate_scatter(hist_v, [v], -cnt, mask=last)
            for v, (cnt, last) in zip(adds, scans_a):
                plsc.addupdate_scatter(hist_v, [v], cnt, mask=last)

            m_l = lane0 & pad_l
            m_r = top_lane & pad_r
            plsc.addupdate_scatter(hist_v, [subs[0]], ones, mask=m_l)
            plsc.addupdate_scatter(hist_v, [x_sub], neg_ones, mask=m_l)
            plsc.addupdate_scatter(hist_v, [x_add], ones, mask=m_r)
            plsc.addupdate_scatter(hist_v, [adds[7]], neg_ones, mask=m_r)

            argmax_store(i + 1)

        pltpu.sync_copy(res_v, out_hbm.at[wid])

    return k(lbl_flat)


def kernel(lbl, W):
    del W
    res = _label_argmax_sc(lbl.reshape(-1))
    res = res.reshape(BATCH, WORKERS_PER_BATCH, RES_PAD)
    return jnp.concatenate(
        [res[:, 0, :64], res[:, 1, :64], res[:, 2, :64], res[:, 3, :65]],
        axis=1)
